# baseline (speedup 1.0000x reference)
import jax
import jax.numpy as jnp
import numpy as np
from jax.sharding import Mesh, PartitionSpec as P
from jax.experimental import pallas as pl
from jax.experimental.pallas import tpu as pltpu

try:
    from jax import shard_map as _shard_map

    def _smap(f, mesh, in_specs, out_specs):
        return _shard_map(f, mesh=mesh, in_specs=in_specs,
                          out_specs=out_specs, check_vma=False)
except ImportError:
    from jax.experimental.shard_map import shard_map as _shard_map_old

    def _smap(f, mesh, in_specs, out_specs):
        return _shard_map_old(f, mesh=mesh, in_specs=in_specs,
                              out_specs=out_specs, check_rep=False)


def _mlp3_kernel(x_ref, w1_ref, b1_ref, w2_ref, b2_ref, w3_ref, b3_ref,
                 o_ref, xb, h1b, h2b):
    n = b1_ref.shape[1]
    tnc = w1_ref.shape[1]
    nc = n // tnc
    c = pl.program_id(1)

    @pl.when(c == 0)
    def _cast_x():
        xb[...] = x_ref[...].astype(jnp.bfloat16)

    @pl.when(c < nc)
    def _layer1_chunk():
        wc = w1_ref[...].astype(jnp.bfloat16)
        acc = jnp.dot(xb[...], wc, preferred_element_type=jnp.float32)
        off = c * tnc
        h1b[:, pl.ds(off, tnc)] = jnp.maximum(
            acc + b1_ref[:, pl.ds(off, tnc)], 0.0).astype(jnp.bfloat16)

    @pl.when(c >= nc)
    def _layer2_chunk():
        wc = w2_ref[...].astype(jnp.bfloat16)
        acc = jnp.dot(h1b[...], wc, preferred_element_type=jnp.float32)
        off = (c - nc) * tnc
        h2b[:, pl.ds(off, tnc)] = jnp.maximum(
            acc + b2_ref[:, pl.ds(off, tnc)], 0.0).astype(jnp.bfloat16)

    @pl.when(c == 2 * nc - 1)
    def _final():
        w3c = w3_ref[...].astype(jnp.bfloat16)
        y = jnp.dot(h2b[...], w3c, preferred_element_type=jnp.float32)
        o_ref[...] = y + b3_ref[...]


def _mlp3(x, w1, b1r, w2, b2r, w3, b3r, *, tm, tnc):
    M, K = x.shape
    N = w1.shape[1]
    O = w3.shape[1]
    nc = N // tnc
    flops = 2 * M * K * N + 2 * M * N * N + 2 * M * N * O
    bytes_accessed = 4 * (M * K + K * N + N * N + N * O + M * O)

    return pl.pallas_call(
        _mlp3_kernel,
        out_shape=jax.ShapeDtypeStruct((M, O), jnp.float32),
        grid=(M // tm, 2 * nc),
        in_specs=[
            pl.BlockSpec((tm, K), lambda i, c: (i, 0)),
            pl.BlockSpec((K, tnc), lambda i, c: (0, jnp.minimum(c, nc - 1))),
            pl.BlockSpec((1, N), lambda i, c: (0, 0)),
            pl.BlockSpec((K, tnc), lambda i, c: (0, jnp.maximum(c - nc, 0))),
            pl.BlockSpec((1, N), lambda i, c: (0, 0)),
            pl.BlockSpec((N, O), lambda i, c: (0, 0)),
            pl.BlockSpec((1, O), lambda i, c: (0, 0)),
        ],
        out_specs=pl.BlockSpec((tm, O), lambda i, c: (i, 0)),
        scratch_shapes=[
            pltpu.VMEM((tm, K), jnp.bfloat16),
            pltpu.VMEM((tm, N), jnp.bfloat16),
            pltpu.VMEM((tm, N), jnp.bfloat16),
        ],
        compiler_params=pltpu.CompilerParams(
            dimension_semantics=("arbitrary", "arbitrary"),
        ),
        cost_estimate=pl.CostEstimate(
            flops=flops, transcendentals=0, bytes_accessed=bytes_accessed
        ),
    )(x, w1, b1r, w2, b2r, w3, b3r)


def _fwd(x, w1, b1, w2, b2, w3, b3):
    M = x.shape[0]
    N = w1.shape[1]
    O = w3.shape[1]
    tm = M if M <= 1024 else max(M // 2, 8)
    tnc = min(512, max(N // 2, 128))
    return _mlp3(x, w1, b1.reshape(1, N), w2, b2.reshape(1, N),
                 w3, b3.reshape(1, O), tm=tm, tnc=tnc)


@jax.jit
def kernel(x, w1, b1, w2, b2, w3, b3):
    M = x.shape[0]
    devs = jax.devices()
    if len(devs) >= 2 and M % 2 == 0 and M >= 16:
        mesh = Mesh(np.asarray(devs[:2]), ("d",))
        fn = _smap(
            _fwd, mesh,
            (P("d", None), P(None, None), P(None), P(None, None), P(None),
             P(None, None), P(None)),
            P("d", None),
        )
        return fn(x, w1, b1, w2, b2, w3, b3)
    return _fwd(x, w1, b1, w2, b2, w3, b3)


# speedup vs baseline: 9.8952x; 9.8952x over previous
import jax
import jax.numpy as jnp
from jax.experimental import pallas as pl
from jax.experimental.pallas import tpu as pltpu


def _mlp3_kernel(x_ref, w1_ref, b1_ref, w2_ref, b2_ref, w3_ref, b3_ref,
                 o_ref, w1b, w2b, xb, h1b, h2b):
    n = b1_ref.shape[1]
    tnc = w1_ref.shape[1]
    nc = n // tnc
    s = pl.program_id(0)

    def finish(h2full):
        w3c = w3_ref[...].astype(jnp.bfloat16)
        y = jnp.dot(h2full, w3c, preferred_element_type=jnp.float32)
        o_ref[...] = y + b3_ref[...]

    @pl.when(s == 0)
    def _cast_x0():
        xb[...] = x_ref[...].astype(jnp.bfloat16)

    @pl.when(s < nc)
    def _stream_w1_chunk():
        c = s
        sl = pl.ds(c * tnc, tnc)
        wc = w1_ref[...].astype(jnp.bfloat16)
        w1b[:, sl] = wc
        acc = jnp.dot(xb[...], wc, preferred_element_type=jnp.float32)
        h1b[:, sl] = jnp.maximum(acc + b1_ref[:, sl], 0.0).astype(jnp.bfloat16)

    @pl.when((s >= nc) & (s < 2 * nc))
    def _stream_w2_chunk():
        c = s - nc
        sl = pl.ds(c * tnc, tnc)
        wc = w2_ref[...].astype(jnp.bfloat16)
        w2b[:, sl] = wc
        acc = jnp.dot(h1b[...], wc, preferred_element_type=jnp.float32)
        h2b[:, sl] = jnp.maximum(acc + b2_ref[:, sl], 0.0).astype(jnp.bfloat16)

    @pl.when(s == 2 * nc - 1)
    def _row0_out():
        finish(h2b[...])

    @pl.when(s >= 2 * nc)
    def _later_rows():
        xr = x_ref[...].astype(jnp.bfloat16)
        for c in range(nc):
            sl = pl.ds(c * tnc, tnc)
            acc = jnp.dot(xr, w1b[:, sl], preferred_element_type=jnp.float32)
            h1b[:, sl] = jnp.maximum(acc + b1_ref[:, sl], 0.0).astype(jnp.bfloat16)
        for c in range(nc):
            sl = pl.ds(c * tnc, tnc)
            acc = jnp.dot(h1b[...], w2b[:, sl], preferred_element_type=jnp.float32)
            h2b[:, sl] = jnp.maximum(acc + b2_ref[:, sl], 0.0).astype(jnp.bfloat16)
        finish(h2b[...])


def _mlp3(x, w1, b1r, w2, b2r, w3, b3r, *, tm, tnc):
    M, K = x.shape
    N = w1.shape[1]
    O = w3.shape[1]
    nc = N // tnc
    nrows = M // tm
    nsteps = 2 * nc + (nrows - 1)
    flops = 2 * M * K * N + 2 * M * N * N + 2 * M * N * O
    bytes_accessed = 4 * (M * K + K * N + N * N + N * O + M * O)

    row_of = lambda s: jnp.maximum(s - (2 * nc - 1), 0)
    return pl.pallas_call(
        _mlp3_kernel,
        out_shape=jax.ShapeDtypeStruct((M, O), jnp.float32),
        grid=(nsteps,),
        in_specs=[
            pl.BlockSpec((tm, K), lambda s: (row_of(s), 0)),
            pl.BlockSpec((K, tnc), lambda s: (0, jnp.minimum(s, nc - 1))),
            pl.BlockSpec((1, N), lambda s: (0, 0)),
            pl.BlockSpec((K, tnc),
                         lambda s: (0, jnp.clip(s - nc, 0, nc - 1))),
            pl.BlockSpec((1, N), lambda s: (0, 0)),
            pl.BlockSpec((N, O), lambda s: (0, 0)),
            pl.BlockSpec((1, O), lambda s: (0, 0)),
        ],
        out_specs=pl.BlockSpec((tm, O), lambda s: (row_of(s), 0)),
        scratch_shapes=[
            pltpu.VMEM((K, N), jnp.bfloat16),
            pltpu.VMEM((N, N), jnp.bfloat16),
            pltpu.VMEM((tm, K), jnp.bfloat16),
            pltpu.VMEM((tm, N), jnp.bfloat16),
            pltpu.VMEM((tm, N), jnp.bfloat16),
        ],
        compiler_params=pltpu.CompilerParams(
            dimension_semantics=("arbitrary",),
        ),
        cost_estimate=pl.CostEstimate(
            flops=flops, transcendentals=0, bytes_accessed=bytes_accessed
        ),
    )(x, w1, b1r, w2, b2r, w3, b3r)


@jax.jit
def kernel(x, w1, b1, w2, b2, w3, b3):
    M = x.shape[0]
    N = w1.shape[1]
    O = w3.shape[1]
    tm = min(512, max(M // 4, 8))
    tnc = min(512, max(N // 2, 128))
    return _mlp3(x, w1, b1.reshape(1, N), w2, b2.reshape(1, N),
                 w3, b3.reshape(1, O), tm=tm, tnc=tnc)
